# baseline (speedup 1.0000x reference)
import jax
import jax.numpy as jnp
import numpy as np
from jax import lax
from jax.experimental import pallas as pl
from jax.experimental.pallas import tpu as pltpu
from jax.experimental.pallas import tpu_sc as plsc

N = 10000
E = 320000
D = 128
H = 8
DH = 16
NEG_SLOPE = 0.2

NC = 2
NS = 16
NW = NC * NS
EPW = E // NW
CH = 40
NCHUNK = EPW // CH
NP = 10240
NPD = NP // 8
NT = NP + NPD
RPT = NT // NS


def _qkv_body(h_ref, w_ref, q_ref, kv_ref):
    prod = jnp.dot(h_ref[...], w_ref[...], preferred_element_type=jnp.float32)
    q_ref[...] = prod[:, :D]
    kv_ref[...] = prod[:, D:]


def _qkv(h, wqkv):
    bn = 1000
    grid = N // bn
    return pl.pallas_call(
        _qkv_body,
        grid=(grid,),
        in_specs=[
            pl.BlockSpec((bn, D), lambda i: (i, 0)),
            pl.BlockSpec((D, 3 * D), lambda i: (0, 0)),
        ],
        out_specs=[
            pl.BlockSpec((bn, D), lambda i: (i, 0)),
            pl.BlockSpec((bn, 2 * D), lambda i: (i, 0)),
        ],
        out_shape=[
            jax.ShapeDtypeStruct((N, D), jnp.float32),
            jax.ShapeDtypeStruct((N, 2 * D), jnp.float32),
        ],
    )(h, wqkv)


def _sc_body(q_hbm, kv_hbm, src_hbm, dst_hbm, acc_hbm,
             src_v, dst_v, dstp_v, idx2_v, qd_v, kvs_v, msx_v, acc_s,
             sem_i, sem_g, sem_s0, sem_s1):
    cid = lax.axis_index("c")
    sid = lax.axis_index("s")
    wid = sid * NC + cid

    z16 = jnp.zeros((16,), jnp.float32)

    def zero_vec_body(i, carry):
        for j in range(D // 16):
            msx_v[0, i, pl.ds(j * 16, 16)] = z16
        return carry

    lax.fori_loop(0, 2 * CH, zero_vec_body, 0, unroll=False)

    nblk = RPT // (2 * CH)

    def zero_spmem_body(i, carry):
        off = pl.multiple_of(sid * RPT + i * 2 * CH, 8)
        pltpu.sync_copy(msx_v.at[0], acc_s.at[pl.ds(off, 2 * CH)])
        return carry

    lax.fori_loop(0, nblk, zero_spmem_body, 0, unroll=False)
    plsc.subcore_barrier()

    lanes = lax.iota(jnp.int32, 16)
    perms = [lanes ^ stp for stp in (8, 4, 2, 1)]
    dnums = lax.GatherDimensionNumbers(
        offset_dims=(), collapsed_slice_dims=(0,), start_index_map=(0,))

    def shuf(v, perm):
        return lax.gather(v, perm[:, None], dimension_numbers=dnums,
                          slice_sizes=(1,),
                          mode=lax.GatherScatterMode.PROMISE_IN_BOUNDS)

    base0 = pl.multiple_of(wid * EPW, 8)
    pltpu.async_copy(src_hbm.at[pl.ds(base0, CH)], src_v.at[0], sem_i)
    pltpu.async_copy(dst_hbm.at[pl.ds(base0, CH)], dst_v.at[0], sem_i)

    def chunk_body(c, carry):
        pb = c & 1
        pltpu.make_async_copy(src_hbm.at[pl.ds(base0, CH)], src_v.at[pb],
                              sem_i).wait()
        pltpu.make_async_copy(dst_hbm.at[pl.ds(base0, CH)], dst_v.at[pb],
                              sem_i).wait()
        cn = jnp.minimum(c + 1, NCHUNK - 1)
        basen = pl.multiple_of(wid * EPW, 8) + cn * CH
        pltpu.async_copy(src_hbm.at[pl.ds(basen, CH)], src_v.at[1 - pb], sem_i)
        pltpu.async_copy(dst_hbm.at[pl.ds(basen, CH)], dst_v.at[1 - pb], sem_i)
        gq = pltpu.async_copy(q_hbm.at[dst_v.at[pb]], qd_v, sem_g)
        gkv = pltpu.async_copy(kv_hbm.at[src_v.at[pb]], kvs_v, sem_g)
        @pl.when(jnp.logical_and(c >= 2, pb == 0))
        def _():
            pltpu.make_async_copy(msx_v.at[0], acc_s.at[idx2_v.at[0]],
                                  sem_s0).wait()

        @pl.when(jnp.logical_and(c >= 2, pb == 1))
        def _():
            pltpu.make_async_copy(msx_v.at[1], acc_s.at[idx2_v.at[1]],
                                  sem_s1).wait()
        for g in (0, 16, CH - 16):
            dv = dst_v[pb, pl.ds(g, 16)]
            idx2_v[pb, pl.ds(g, 16)] = dv
            idx2_v[pb, pl.ds(CH + g, 16)] = (dv >> 3) + NP
            dstp_v[pl.ds(g, 16)] = dv
        gq.wait()
        gkv.wait()

        def group_body(gi, carry2):
            goff = pl.multiple_of(gi * 8, 8)
            dgvf = (dstp_v[pl.ds(goff, 16)] & 7).astype(jnp.float32)
            for j in range(8):
                e = goff + j
                exrow = z16
                for h in range(H):
                    qv = qd_v[e, pl.ds(h * DH, DH)]
                    kvvec = kvs_v[e, pl.ds(h * DH, DH)]
                    p = qv * kvvec
                    for perm in perms:
                        p = p + shuf(p, perm)
                    s = jnp.where(p >= 0, p, NEG_SLOPE * p)
                    ex = jnp.exp(s)
                    vv = kvs_v[e, pl.ds(D + h * DH, DH)]
                    msx_v[pb, e, pl.ds(h * DH, DH)] = vv * ex
                    exrow = jnp.where(lanes == h, ex, exrow)
                dm = jnp.where(lanes == j, dgvf, 0.0)
                for perm in perms:
                    dm = dm + shuf(dm, perm)
                for jj in range(8):
                    msx_v[pb, CH + e, pl.ds(jj * 16, 16)] = jnp.where(
                        dm == float(jj), exrow, z16)
            return carry2

        lax.fori_loop(0, CH // 8, group_body, 0, unroll=False)
        @pl.when(pb == 0)
        def _():
            pltpu.async_copy(msx_v.at[0], acc_s.at[idx2_v.at[0]], sem_s0,
                             add=True)

        @pl.when(pb == 1)
        def _():
            pltpu.async_copy(msx_v.at[1], acc_s.at[idx2_v.at[1]], sem_s1,
                             add=True)
        return carry

    lax.fori_loop(0, NCHUNK, chunk_body, 0, unroll=False)
    pltpu.make_async_copy(msx_v.at[0], acc_s.at[idx2_v.at[0]], sem_s0).wait()
    pltpu.make_async_copy(msx_v.at[1], acc_s.at[idx2_v.at[1]], sem_s1).wait()
    pltpu.make_async_copy(src_hbm.at[pl.ds(base0, CH)], src_v.at[0],
                          sem_i).wait()
    pltpu.make_async_copy(dst_hbm.at[pl.ds(base0, CH)], dst_v.at[0],
                          sem_i).wait()
    plsc.subcore_barrier()

    def readout_body(i, carry):
        off = pl.multiple_of(sid * RPT + i * 2 * CH, 8)
        pltpu.sync_copy(acc_s.at[pl.ds(off, 2 * CH)], msx_v.at[0])
        pltpu.sync_copy(msx_v.at[0], acc_hbm.at[cid, pl.ds(off, 2 * CH)])
        return carry

    lax.fori_loop(0, nblk, readout_body, 0, unroll=False)


def _sc_edges(q, kv, src, dst):
    mesh = plsc.VectorSubcoreMesh(core_axis_name="c", subcore_axis_name="s")
    fn = pl.kernel(
        _sc_body,
        out_type=jax.ShapeDtypeStruct((NC, NT, D), jnp.float32),
        mesh=mesh,
        scratch_types=[
            pltpu.VMEM((2, CH), jnp.int32),
            pltpu.VMEM((2, CH), jnp.int32),
            pltpu.VMEM((CH + 16,), jnp.int32),
            pltpu.VMEM((2, 2 * CH), jnp.int32),
            pltpu.VMEM((CH, D), jnp.float32),
            pltpu.VMEM((CH, 2 * D), jnp.float32),
            pltpu.VMEM((2, 2 * CH, D), jnp.float32),
            pltpu.VMEM_SHARED((NT, D), jnp.float32),
            pltpu.SemaphoreType.DMA,
            pltpu.SemaphoreType.DMA,
            pltpu.SemaphoreType.DMA,
            pltpu.SemaphoreType.DMA,
        ],
    )
    return fn(q, kv, src, dst)


def _final_body(h_ref, wres_ref, num_ref, den_ref, bexp_ref, out_ref):
    res = jnp.dot(h_ref[...], wres_ref[...], preferred_element_type=jnp.float32)
    num = num_ref[0] + num_ref[1]
    den = den_ref[0] + den_ref[1]
    dexp = jnp.dot(den, bexp_ref[...], preferred_element_type=jnp.float32)
    dexp = jnp.where(dexp == 0.0, 1.0, dexp)
    out_ref[...] = res + num / dexp


def _final(h, wres, num, den, bexp):
    bn = 1000
    grid = N // bn
    return pl.pallas_call(
        _final_body,
        grid=(grid,),
        in_specs=[
            pl.BlockSpec((bn, D), lambda i: (i, 0)),
            pl.BlockSpec((D, D), lambda i: (0, 0)),
            pl.BlockSpec((NC, bn, D), lambda i: (0, i, 0)),
            pl.BlockSpec((NC, bn, 16), lambda i: (0, i, 0)),
            pl.BlockSpec((16, D), lambda i: (0, 0)),
        ],
        out_specs=pl.BlockSpec((bn, D), lambda i: (i, 0)),
        out_shape=jax.ShapeDtypeStruct((N, D), jnp.float32),
    )(h, wres, num, den, bexp)


def kernel(h, edge_features, edge_index, Wq, Wk, Wv, We, Wres):
    del edge_features, We
    src = edge_index[0]
    dst = edge_index[1]
    wqkv = jnp.concatenate([Wq, Wk, Wv], axis=1)
    bexp = np.zeros((16, D), np.float32)
    for hh in range(H):
        bexp[hh, hh * DH:(hh + 1) * DH] = 1.0
    bexp = jnp.asarray(bexp)

    q, kv = _qkv(h, wqkv)
    acc = _sc_edges(q, kv, src, dst)
    num = acc[:, :NP, :]
    den = acc[:, NP:, :].reshape(NC, NP, 16)
    return _final(h, Wres, num, den, bexp)


# speedup vs baseline: 1.2368x; 1.2368x over previous
import jax
import jax.numpy as jnp
import numpy as np
from jax import lax
from jax.experimental import pallas as pl
from jax.experimental.pallas import tpu as pltpu
from jax.experimental.pallas import tpu_sc as plsc

N = 10000
E = 320000
D = 128
H = 8
DH = 16
NEG_SLOPE = 0.2

NC = 2
NS = 16
NW = NC * NS
EPW = E // NW
CH = 40
NCHUNK = EPW // CH
NP = 10240
NPD = NP // 8
NT = NP + NPD
RPT = NT // NS


def _qkv_body(h_ref, w_ref, q_ref, kv_ref):
    prod = jnp.dot(h_ref[...], w_ref[...], preferred_element_type=jnp.float32)
    q_ref[...] = prod[:, :D]
    kv_ref[...] = prod[:, D:]


def _qkv(h, wqkv):
    bn = 1000
    grid = N // bn
    return pl.pallas_call(
        _qkv_body,
        grid=(grid,),
        in_specs=[
            pl.BlockSpec((bn, D), lambda i: (i, 0)),
            pl.BlockSpec((D, 3 * D), lambda i: (0, 0)),
        ],
        out_specs=[
            pl.BlockSpec((bn, D), lambda i: (i, 0)),
            pl.BlockSpec((bn, 2 * D), lambda i: (i, 0)),
        ],
        out_shape=[
            jax.ShapeDtypeStruct((N, D), jnp.float32),
            jax.ShapeDtypeStruct((N, 2 * D), jnp.float32),
        ],
    )(h, wqkv)


def _sc_body(q_hbm, kv_hbm, src_hbm, dst_hbm, acc_hbm,
             src_v, dst_v, dstp_v, idx2_v, qd_v, kvs_v, msx_v, acc_s,
             sem_i, sem_g, sem_s):
    cid = lax.axis_index("c")
    sid = lax.axis_index("s")
    wid = sid * NC + cid

    z16 = jnp.zeros((16,), jnp.float32)

    def zero_vec_body(i, carry):
        for j in range(D // 16):
            msx_v[i, pl.ds(j * 16, 16)] = z16
        return carry

    lax.fori_loop(0, 2 * CH, zero_vec_body, 0, unroll=False)

    nblk = RPT // (2 * CH)

    def zero_spmem_body(i, carry):
        off = pl.multiple_of(sid * RPT + i * 2 * CH, 8)
        pltpu.sync_copy(msx_v, acc_s.at[pl.ds(off, 2 * CH)])
        return carry

    lax.fori_loop(0, nblk, zero_spmem_body, 0, unroll=False)
    plsc.subcore_barrier()

    lanes = lax.iota(jnp.int32, 16)
    perms = [lanes ^ stp for stp in (8, 4, 2, 1)]
    dnums = lax.GatherDimensionNumbers(
        offset_dims=(), collapsed_slice_dims=(0,), start_index_map=(0,))

    def shuf(v, perm):
        return lax.gather(v, perm[:, None], dimension_numbers=dnums,
                          slice_sizes=(1,),
                          mode=lax.GatherScatterMode.PROMISE_IN_BOUNDS)

    base0 = pl.multiple_of(wid * EPW, 8)
    pltpu.async_copy(src_hbm.at[pl.ds(base0, CH)], src_v.at[0], sem_i)
    pltpu.async_copy(dst_hbm.at[pl.ds(base0, CH)], dst_v.at[0], sem_i)

    def chunk_body(c, carry):
        pb = c & 1
        pltpu.make_async_copy(src_hbm.at[pl.ds(base0, CH)], src_v.at[pb],
                              sem_i).wait()
        pltpu.make_async_copy(dst_hbm.at[pl.ds(base0, CH)], dst_v.at[pb],
                              sem_i).wait()
        cn = jnp.minimum(c + 1, NCHUNK - 1)
        basen = pl.multiple_of(wid * EPW, 8) + cn * CH
        pltpu.async_copy(src_hbm.at[pl.ds(basen, CH)], src_v.at[1 - pb], sem_i)
        pltpu.async_copy(dst_hbm.at[pl.ds(basen, CH)], dst_v.at[1 - pb], sem_i)
        gq = pltpu.async_copy(q_hbm.at[dst_v.at[pb]], qd_v, sem_g)
        gkv = pltpu.async_copy(kv_hbm.at[src_v.at[pb]], kvs_v, sem_g)
        gq.wait()
        gkv.wait()
        @pl.when(c > 0)
        def _():
            pltpu.make_async_copy(msx_v, acc_s.at[idx2_v], sem_s).wait()
        for g in (0, 16, CH - 16):
            dv = dst_v[pb, pl.ds(g, 16)]
            idx2_v[pl.ds(g, 16)] = dv
            idx2_v[pl.ds(CH + g, 16)] = (dv >> 3) + NP
            dstp_v[pl.ds(g, 16)] = dv

        def group_body(gi, carry2):
            goff = pl.multiple_of(gi * 8, 8)
            dgvf = (dstp_v[pl.ds(goff, 16)] & 7).astype(jnp.float32)
            for j in range(8):
                e = goff + j
                exrow = z16
                for h in range(H):
                    qv = qd_v[e, pl.ds(h * DH, DH)]
                    kvvec = kvs_v[e, pl.ds(h * DH, DH)]
                    p = qv * kvvec
                    for perm in perms:
                        p = p + shuf(p, perm)
                    s = jnp.where(p >= 0, p, NEG_SLOPE * p)
                    ex = jnp.exp(s)
                    vv = kvs_v[e, pl.ds(D + h * DH, DH)]
                    msx_v[e, pl.ds(h * DH, DH)] = vv * ex
                    exrow = jnp.where(lanes == h, ex, exrow)
                dm = jnp.where(lanes == j, dgvf, 0.0)
                for perm in perms:
                    dm = dm + shuf(dm, perm)
                for jj in range(8):
                    msx_v[CH + e, pl.ds(jj * 16, 16)] = jnp.where(
                        dm == float(jj), exrow, z16)
            return carry2

        lax.fori_loop(0, CH // 8, group_body, 0, unroll=False)
        pltpu.async_copy(msx_v, acc_s.at[idx2_v], sem_s, add=True)
        return carry

    lax.fori_loop(0, NCHUNK, chunk_body, 0, unroll=False)
    pltpu.make_async_copy(msx_v, acc_s.at[idx2_v], sem_s).wait()
    pltpu.make_async_copy(src_hbm.at[pl.ds(base0, CH)], src_v.at[0],
                          sem_i).wait()
    pltpu.make_async_copy(dst_hbm.at[pl.ds(base0, CH)], dst_v.at[0],
                          sem_i).wait()
    plsc.subcore_barrier()

    def readout_body(i, carry):
        off = pl.multiple_of(sid * RPT + i * 2 * CH, 8)
        pltpu.sync_copy(acc_s.at[pl.ds(off, 2 * CH)], msx_v)
        pltpu.sync_copy(msx_v, acc_hbm.at[cid, pl.ds(off, 2 * CH)])
        return carry

    lax.fori_loop(0, nblk, readout_body, 0, unroll=False)


def _sc_edges(q, kv, src, dst):
    mesh = plsc.VectorSubcoreMesh(core_axis_name="c", subcore_axis_name="s")
    fn = pl.kernel(
        _sc_body,
        out_type=jax.ShapeDtypeStruct((NC, NT, D), jnp.float32),
        mesh=mesh,
        scratch_types=[
            pltpu.VMEM((2, CH), jnp.int32),
            pltpu.VMEM((2, CH), jnp.int32),
            pltpu.VMEM((CH + 16,), jnp.int32),
            pltpu.VMEM((2 * CH,), jnp.int32),
            pltpu.VMEM((CH, D), jnp.float32),
            pltpu.VMEM((CH, 2 * D), jnp.float32),
            pltpu.VMEM((2 * CH, D), jnp.float32),
            pltpu.VMEM_SHARED((NT, D), jnp.float32),
            pltpu.SemaphoreType.DMA,
            pltpu.SemaphoreType.DMA,
            pltpu.SemaphoreType.DMA,
        ],
    )
    return fn(q, kv, src, dst)


def _final_body(h_ref, wres_ref, num_ref, den_ref, bexp_ref, out_ref):
    res = jnp.dot(h_ref[...], wres_ref[...], preferred_element_type=jnp.float32)
    num = num_ref[0] + num_ref[1]
    den = den_ref[0] + den_ref[1]
    dexp = jnp.dot(den, bexp_ref[...], preferred_element_type=jnp.float32)
    dexp = jnp.where(dexp == 0.0, 1.0, dexp)
    out_ref[...] = res + num / dexp


def _final(h, wres, num, den, bexp):
    bn = 1000
    grid = N // bn
    return pl.pallas_call(
        _final_body,
        grid=(grid,),
        in_specs=[
            pl.BlockSpec((bn, D), lambda i: (i, 0)),
            pl.BlockSpec((D, D), lambda i: (0, 0)),
            pl.BlockSpec((NC, bn, D), lambda i: (0, i, 0)),
            pl.BlockSpec((NC, bn, 16), lambda i: (0, i, 0)),
            pl.BlockSpec((16, D), lambda i: (0, 0)),
        ],
        out_specs=pl.BlockSpec((bn, D), lambda i: (i, 0)),
        out_shape=jax.ShapeDtypeStruct((N, D), jnp.float32),
    )(h, wres, num, den, bexp)


def kernel(h, edge_features, edge_index, Wq, Wk, Wv, We, Wres):
    del edge_features, We
    src = edge_index[0]
    dst = edge_index[1]
    wqkv = jnp.concatenate([Wq, Wk, Wv], axis=1)
    bexp = np.zeros((16, D), np.float32)
    for hh in range(H):
        bexp[hh, hh * DH:(hh + 1) * DH] = 1.0
    bexp = jnp.asarray(bexp)

    q, kv = _qkv(h, wqkv)
    acc = _sc_edges(q, kv, src, dst)
    num = acc[:, :NP, :]
    den = acc[:, NP:, :].reshape(NC, NP, 16)
    return _final(h, Wres, num, den, bexp)


# speedup vs baseline: 2.5921x; 2.0958x over previous
import jax
import jax.numpy as jnp
import numpy as np
from jax import lax
from jax.experimental import pallas as pl
from jax.experimental.pallas import tpu as pltpu
from jax.experimental.pallas import tpu_sc as plsc

N = 10000
E = 320000
D = 128
H = 8
DH = 16
NEG_SLOPE = 0.2

NC = 2
NS = 16
NW = NC * NS
EPW = E // NW
CH = 40
NCHUNK = EPW // CH
NP = 10240
NPD = NP // 8
NT = NP + NPD
RPT = NT // NS


def _qkv_body(h_ref, w_ref, q_ref, kv_ref):
    prod = jnp.dot(h_ref[...], w_ref[...], preferred_element_type=jnp.float32)
    q_ref[...] = prod[:, :D]
    kv_ref[...] = prod[:, D:]


def _qkv(h, wqkv):
    bn = 1000
    grid = N // bn
    return pl.pallas_call(
        _qkv_body,
        grid=(grid,),
        in_specs=[
            pl.BlockSpec((bn, D), lambda i: (i, 0)),
            pl.BlockSpec((D, 3 * D), lambda i: (0, 0)),
        ],
        out_specs=[
            pl.BlockSpec((bn, D), lambda i: (i, 0)),
            pl.BlockSpec((bn, 2 * D), lambda i: (i, 0)),
        ],
        out_shape=[
            jax.ShapeDtypeStruct((N, D), jnp.float32),
            jax.ShapeDtypeStruct((N, 2 * D), jnp.float32),
        ],
    )(h, wqkv)


def _sc_body(q_hbm, kv_hbm, src_hbm, dst_hbm, acc_hbm,
             src_v, dst_v, dstp_v, idx2_v, qd_v, kvs_v, msx_v, acc_s,
             sem_i, sem_g, sem_s):
    cid = lax.axis_index("c")
    sid = lax.axis_index("s")
    wid = sid * NC + cid

    z16 = jnp.zeros((16,), jnp.float32)

    def zero_vec_body(i, carry):
        for j in range(D // 16):
            msx_v[i, pl.ds(j * 16, 16)] = z16
        return carry

    lax.fori_loop(0, 2 * CH, zero_vec_body, 0, unroll=False)

    nblk = RPT // (2 * CH)

    def zero_spmem_body(i, carry):
        off = pl.multiple_of(sid * RPT + i * 2 * CH, 8)
        pltpu.sync_copy(msx_v, acc_s.at[pl.ds(off, 2 * CH)])
        return carry

    lax.fori_loop(0, nblk, zero_spmem_body, 0, unroll=False)
    plsc.subcore_barrier()

    lanes = lax.iota(jnp.int32, 16)
    perms = [lanes ^ stp for stp in (8, 4, 2, 1)]
    dnums = lax.GatherDimensionNumbers(
        offset_dims=(), collapsed_slice_dims=(0,), start_index_map=(0,))

    def shuf(v, perm):
        return lax.gather(v, perm[:, None], dimension_numbers=dnums,
                          slice_sizes=(1,),
                          mode=lax.GatherScatterMode.PROMISE_IN_BOUNDS)

    base0 = pl.multiple_of(wid * EPW, 8)
    pltpu.async_copy(src_hbm.at[pl.ds(base0, CH)], src_v.at[0], sem_i)
    pltpu.async_copy(dst_hbm.at[pl.ds(base0, CH)], dst_v.at[0], sem_i)

    def chunk_body(c, carry):
        pb = c & 1
        pltpu.make_async_copy(src_hbm.at[pl.ds(base0, CH)], src_v.at[pb],
                              sem_i).wait()
        pltpu.make_async_copy(dst_hbm.at[pl.ds(base0, CH)], dst_v.at[pb],
                              sem_i).wait()
        cn = jnp.minimum(c + 1, NCHUNK - 1)
        basen = pl.multiple_of(wid * EPW, 8) + cn * CH
        pltpu.async_copy(src_hbm.at[pl.ds(basen, CH)], src_v.at[1 - pb], sem_i)
        pltpu.async_copy(dst_hbm.at[pl.ds(basen, CH)], dst_v.at[1 - pb], sem_i)
        gq = pltpu.async_copy(q_hbm.at[dst_v.at[pb]], qd_v, sem_g)
        gkv = pltpu.async_copy(kv_hbm.at[src_v.at[pb]], kvs_v, sem_g)
        gq.wait()
        gkv.wait()
        @pl.when(c > 0)
        def _():
            pltpu.make_async_copy(msx_v, acc_s.at[idx2_v], sem_s).wait()
        for g in (0, 16, CH - 16):
            dv = dst_v[pb, pl.ds(g, 16)]
            idx2_v[pl.ds(g, 16)] = dv
            idx2_v[pl.ds(CH + g, 16)] = (dv >> 3) + NP
            dstp_v[pl.ds(g, 16)] = dv

        def group_body(gi, carry2):
            goff = pl.multiple_of(gi * 8, 8)
            dgvf = (dstp_v[pl.ds(goff, 16)] & 7).astype(jnp.float32)
            permA = (lanes + ((lanes >> 2) & 1) * 4) & 15
            permB = (lanes - 8 + ((lanes >> 2) & 1) * 4) & 15
            permC = (4 * (lanes >> 1) + (lanes & 1)) & 15
            permD = (4 * ((lanes >> 1) - 4) + (lanes & 1)) & 15
            permE = (lanes & 7) * 2
            spl = [(lanes & 0) + h for h in range(H)]
            for j in range(8):
                e = goff + j
                t = []
                for h in range(H):
                    qv = qd_v[e, pl.ds(h * DH, DH)]
                    kvvec = kvs_v[e, pl.ds(h * DH, DH)]
                    pr = qv * kvvec
                    t.append(pr + shuf(pr, perms[0]))
                c = [jnp.where(lanes < 8, t[2 * i], t[2 * i + 1])
                     for i in range(4)]
                u = [ci + shuf(ci, perms[1]) for ci in c]
                f = [jnp.where(lanes < 8, shuf(u[2 * i], permA),
                               shuf(u[2 * i + 1], permB)) for i in range(2)]
                w = [fi + shuf(fi, perms[2]) for fi in f]
                g2 = jnp.where(lanes < 8, shuf(w[0], permC),
                               shuf(w[1], permD))
                x = g2 + shuf(g2, perms[3])
                dots = shuf(x, permE)
                s = jnp.where(dots >= 0, dots, NEG_SLOPE * dots)
                exrow = jnp.where(lanes < 8, jnp.exp(s), 0.0)
                for h in range(H):
                    exs = shuf(exrow, spl[h])
                    vv = kvs_v[e, pl.ds(D + h * DH, DH)]
                    msx_v[e, pl.ds(h * DH, DH)] = vv * exs
                dm = jnp.where(lanes == j, dgvf, 0.0)
                for perm in perms:
                    dm = dm + shuf(dm, perm)
                for jj in range(8):
                    msx_v[CH + e, pl.ds(jj * 16, 16)] = jnp.where(
                        dm == float(jj), exrow, z16)
            return carry2

        lax.fori_loop(0, CH // 8, group_body, 0, unroll=False)
        pltpu.async_copy(msx_v, acc_s.at[idx2_v], sem_s, add=True)
        return carry

    lax.fori_loop(0, NCHUNK, chunk_body, 0, unroll=False)
    pltpu.make_async_copy(msx_v, acc_s.at[idx2_v], sem_s).wait()
    pltpu.make_async_copy(src_hbm.at[pl.ds(base0, CH)], src_v.at[0],
                          sem_i).wait()
    pltpu.make_async_copy(dst_hbm.at[pl.ds(base0, CH)], dst_v.at[0],
                          sem_i).wait()
    plsc.subcore_barrier()

    def readout_body(i, carry):
        off = pl.multiple_of(sid * RPT + i * 2 * CH, 8)
        pltpu.sync_copy(acc_s.at[pl.ds(off, 2 * CH)], msx_v)
        pltpu.sync_copy(msx_v, acc_hbm.at[cid, pl.ds(off, 2 * CH)])
        return carry

    lax.fori_loop(0, nblk, readout_body, 0, unroll=False)


def _sc_edges(q, kv, src, dst):
    mesh = plsc.VectorSubcoreMesh(core_axis_name="c", subcore_axis_name="s")
    fn = pl.kernel(
        _sc_body,
        out_type=jax.ShapeDtypeStruct((NC, NT, D), jnp.float32),
        mesh=mesh,
        scratch_types=[
            pltpu.VMEM((2, CH), jnp.int32),
            pltpu.VMEM((2, CH), jnp.int32),
            pltpu.VMEM((CH + 16,), jnp.int32),
            pltpu.VMEM((2 * CH,), jnp.int32),
            pltpu.VMEM((CH, D), jnp.float32),
            pltpu.VMEM((CH, 2 * D), jnp.float32),
            pltpu.VMEM((2 * CH, D), jnp.float32),
            pltpu.VMEM_SHARED((NT, D), jnp.float32),
            pltpu.SemaphoreType.DMA,
            pltpu.SemaphoreType.DMA,
            pltpu.SemaphoreType.DMA,
        ],
    )
    return fn(q, kv, src, dst)


def _final_body(h_ref, wres_ref, num_ref, den_ref, bexp_ref, out_ref):
    res = jnp.dot(h_ref[...], wres_ref[...], preferred_element_type=jnp.float32)
    num = num_ref[0] + num_ref[1]
    den = den_ref[0] + den_ref[1]
    dexp = jnp.dot(den, bexp_ref[...], preferred_element_type=jnp.float32)
    dexp = jnp.where(dexp == 0.0, 1.0, dexp)
    out_ref[...] = res + num / dexp


def _final(h, wres, num, den, bexp):
    bn = 1000
    grid = N // bn
    return pl.pallas_call(
        _final_body,
        grid=(grid,),
        in_specs=[
            pl.BlockSpec((bn, D), lambda i: (i, 0)),
            pl.BlockSpec((D, D), lambda i: (0, 0)),
            pl.BlockSpec((NC, bn, D), lambda i: (0, i, 0)),
            pl.BlockSpec((NC, bn, 16), lambda i: (0, i, 0)),
            pl.BlockSpec((16, D), lambda i: (0, 0)),
        ],
        out_specs=pl.BlockSpec((bn, D), lambda i: (i, 0)),
        out_shape=jax.ShapeDtypeStruct((N, D), jnp.float32),
    )(h, wres, num, den, bexp)


def kernel(h, edge_features, edge_index, Wq, Wk, Wv, We, Wres):
    del edge_features, We
    src = edge_index[0]
    dst = edge_index[1]
    wqkv = jnp.concatenate([Wq, Wk, Wv], axis=1)
    bexp = np.zeros((16, D), np.float32)
    for hh in range(H):
        bexp[hh, hh * DH:(hh + 1) * DH] = 1.0
    bexp = jnp.asarray(bexp)

    q, kv = _qkv(h, wqkv)
    acc = _sc_edges(q, kv, src, dst)
    num = acc[:, :NP, :]
    den = acc[:, NP:, :].reshape(NC, NP, 16)
    return _final(h, Wres, num, den, bexp)
